# baseline (speedup 1.0000x reference)
import functools

import jax
import jax.numpy as jnp
from jax.experimental import pallas as pl
from jax.experimental.pallas import tpu as pltpu
from jax.scipy.linalg import block_diag

ALPHA = 0.1
Q = 1.0 - ALPHA
EPS = 1e-5
BLK = 400
LANES = 128


def _global_stats(part, n_elems):
    s = jnp.sum(part[:, 0, 0])
    ss = jnp.sum(part[:, 0, 1])
    mu = s / n_elems
    var = jnp.maximum(ss / n_elems - mu * mu, 0.0)
    return mu, jax.lax.rsqrt(var + EPS)


def _partial_vec(t):
    s = jnp.sum(t)
    ss = jnp.sum(t * t)
    lane = jax.lax.broadcasted_iota(jnp.int32, (1, 1, LANES), 2)
    return jnp.where(lane == 0, s, jnp.where(lane == 1, ss, 0.0))


def _pernode_bn_relu(t, gamma, beta):
    mu = jnp.mean(t, axis=(1, 2), keepdims=True)
    var = jnp.maximum(jnp.mean(t * t, axis=(1, 2), keepdims=True) - mu * mu, 0.0)
    rs = jax.lax.rsqrt(var + EPS)
    return jnp.maximum(gamma * (t - mu) * rs + beta, 0.0)


def _fold(v, width):
    while v.shape[-1] > width:
        half = v.shape[-1] // 2
        v = v[:, :half] + v[:, half:]
    return v


def _pass1_body(x_ref, nb_ref, w1_ref, w1d_ref, b1_ref, b1d_ref,
                naggp_ref, h_ref, x1p_ref, part_ref):
    B, R, twoD = nb_ref.shape
    H1 = w1_ref.shape[1]
    xb = x_ref[...]
    h = jnp.dot(xb, w1_ref[...], preferred_element_type=jnp.float32) + b1_ref[...]
    nh = jnp.dot(nb_ref[...].reshape(B * R, twoD), w1d_ref[...],
                 preferred_element_type=jnp.float32) + b1d_ref[...]
    nh3 = nh.reshape(B, R, 2 * H1)
    s = _fold(jnp.sum(nh3, axis=1), H1)
    x1p = h + Q * s
    hh = jnp.concatenate([h, h], axis=-1)
    naggp_ref[...] = nh3 + Q * hh[:, None, :]
    h_ref[...] = h
    x1p_ref[...] = x1p
    part_ref[...] = _partial_vec(x1p)


def _pass2_body(naggp_ref, h_ref, x1p_ref, part_ref, w2d_ref, b2d_ref, gb_ref,
                nagg2p_ref, h2_ref, x3p_ref, part2_ref, *, n_elems):
    B, R4, fourH1 = naggp_ref.shape
    H1 = fourH1 // 4
    H2 = w2d_ref.shape[1] // 4
    gamma = gb_ref[0, 0]
    beta = gb_ref[0, 1]
    mu, rs = _global_stats(part_ref[...], n_elems)
    x1 = jnp.maximum(gamma * (x1p_ref[...] - mu) * rs + beta, 0.0)
    naggp = naggp_ref[...]
    nagg = _pernode_bn_relu(naggp, gamma, beta)
    h = h_ref[...]
    x2 = Q * (x1 + _fold(jnp.sum(nagg, axis=1), H1)) + ALPHA * h
    c = Q * x1 - ALPHA * Q * h
    cc = jnp.concatenate([c, c, c, c], axis=-1)
    n2 = Q * nagg + ALPHA * naggp + cc[:, None, :]
    h2 = jnp.dot(x2, w2d_ref[:H1, :H2],
                 preferred_element_type=jnp.float32) + b2d_ref[:, :H2]
    nh2 = jnp.dot(n2.reshape(B * R4, fourH1), w2d_ref[...],
                  preferred_element_type=jnp.float32) + b2d_ref[...]
    nh23 = nh2.reshape(B, R4, 4 * H2)
    s2 = _fold(jnp.sum(nh23, axis=1), H2)
    x3p = h2 + Q * s2
    h2c = jnp.concatenate([h2, h2, h2, h2], axis=-1)
    nagg2p_ref[...] = nh23 + Q * h2c[:, None, :]
    h2_ref[...] = h2
    x3p_ref[...] = x3p
    part2_ref[...] = _partial_vec(x3p)


def _pass3_body(nagg2p_ref, h2_ref, x3p_ref, part_ref, wc_ref, bc_ref, gb_ref,
                out_ref, *, n_elems):
    H2 = h2_ref.shape[-1]
    gamma = gb_ref[0, 0]
    beta = gb_ref[0, 1]
    mu, rs = _global_stats(part_ref[...], n_elems)
    x3 = jnp.maximum(gamma * (x3p_ref[...] - mu) * rs + beta, 0.0)
    nagg2 = _pernode_bn_relu(nagg2p_ref[...], gamma, beta)
    h2 = h2_ref[...]
    x4 = Q * (x3 + _fold(jnp.sum(nagg2, axis=1), H2)) + ALPHA * h2
    x4 = jnp.where(jnp.isnan(x4), 0.0, x4)
    out_ref[...] = jnp.dot(x4, wc_ref[...],
                           preferred_element_type=jnp.float32) + bc_ref[...]


def kernel(x, neighbor, W1, b1, W2, b2, Wc, bc, gamma, beta):
    N, _, D = x.shape
    K = neighbor.shape[1]
    H1 = W1.shape[1]
    H2 = W2.shape[1]
    C = Wc.shape[1]
    B = BLK
    G = N // B
    f32 = jnp.float32

    x2d = x.reshape(N, D)
    nbp = neighbor.reshape(N, K // 2, 2 * D)
    W1d = block_diag(W1, W1)
    b1d = jnp.concatenate([b1, b1]).reshape(1, 2 * H1)
    W2d = block_diag(W2, W2, W2, W2)
    b2d = jnp.concatenate([b2, b2, b2, b2]).reshape(1, 4 * H2)
    b1r = b1.reshape(1, H1)
    bcr = bc.reshape(1, C)
    gb = jnp.concatenate([gamma, beta]).reshape(1, 2)

    params = pltpu.CompilerParams(dimension_semantics=("parallel",))

    naggp, h, x1p, part1 = pl.pallas_call(
        _pass1_body,
        grid=(G,),
        in_specs=[
            pl.BlockSpec((B, D), lambda i: (i, 0)),
            pl.BlockSpec((B, K // 2, 2 * D), lambda i: (i, 0, 0)),
            pl.BlockSpec((D, H1), lambda i: (0, 0)),
            pl.BlockSpec((2 * D, 2 * H1), lambda i: (0, 0)),
            pl.BlockSpec((1, H1), lambda i: (0, 0)),
            pl.BlockSpec((1, 2 * H1), lambda i: (0, 0)),
        ],
        out_specs=[
            pl.BlockSpec((B, K // 2, 2 * H1), lambda i: (i, 0, 0)),
            pl.BlockSpec((B, H1), lambda i: (i, 0)),
            pl.BlockSpec((B, H1), lambda i: (i, 0)),
            pl.BlockSpec((1, 1, LANES), lambda i: (i, 0, 0)),
        ],
        out_shape=[
            jax.ShapeDtypeStruct((N, K // 2, 2 * H1), f32),
            jax.ShapeDtypeStruct((N, H1), f32),
            jax.ShapeDtypeStruct((N, H1), f32),
            jax.ShapeDtypeStruct((G, 1, LANES), f32),
        ],
        compiler_params=params,
    )(x2d, nbp, W1, W1d, b1r, b1d)

    naggp4 = naggp.reshape(N, K // 4, 4 * H1)

    nagg2p, h2, x3p, part2 = pl.pallas_call(
        functools.partial(_pass2_body, n_elems=float(N * H1)),
        grid=(G,),
        in_specs=[
            pl.BlockSpec((B, K // 4, 4 * H1), lambda i: (i, 0, 0)),
            pl.BlockSpec((B, H1), lambda i: (i, 0)),
            pl.BlockSpec((B, H1), lambda i: (i, 0)),
            pl.BlockSpec((G, 1, LANES), lambda i: (0, 0, 0)),
            pl.BlockSpec((4 * H1, 4 * H2), lambda i: (0, 0)),
            pl.BlockSpec((1, 4 * H2), lambda i: (0, 0)),
            pl.BlockSpec((1, 2), lambda i: (0, 0)),
        ],
        out_specs=[
            pl.BlockSpec((B, K // 4, 4 * H2), lambda i: (i, 0, 0)),
            pl.BlockSpec((B, H2), lambda i: (i, 0)),
            pl.BlockSpec((B, H2), lambda i: (i, 0)),
            pl.BlockSpec((1, 1, LANES), lambda i: (i, 0, 0)),
        ],
        out_shape=[
            jax.ShapeDtypeStruct((N, K // 4, 4 * H2), f32),
            jax.ShapeDtypeStruct((N, H2), f32),
            jax.ShapeDtypeStruct((N, H2), f32),
            jax.ShapeDtypeStruct((G, 1, LANES), f32),
        ],
        compiler_params=params,
    )(naggp4, h, x1p, part1, W2d, b2d, gb)

    out = pl.pallas_call(
        functools.partial(_pass3_body, n_elems=float(N * H2)),
        grid=(G,),
        in_specs=[
            pl.BlockSpec((B, K // 4, 4 * H2), lambda i: (i, 0, 0)),
            pl.BlockSpec((B, H2), lambda i: (i, 0)),
            pl.BlockSpec((B, H2), lambda i: (i, 0)),
            pl.BlockSpec((G, 1, LANES), lambda i: (0, 0, 0)),
            pl.BlockSpec((H2, C), lambda i: (0, 0)),
            pl.BlockSpec((1, C), lambda i: (0, 0)),
            pl.BlockSpec((1, 2), lambda i: (0, 0)),
        ],
        out_specs=pl.BlockSpec((B, C), lambda i: (i, 0)),
        out_shape=jax.ShapeDtypeStruct((N, C), f32),
        compiler_params=params,
    )(nagg2p, h2, x3p, part2, Wc, bcr, gb)

    return out


# speedup vs baseline: 2.6287x; 2.6287x over previous
import functools

import jax
import jax.numpy as jnp
from jax.experimental import pallas as pl
from jax.experimental.pallas import tpu as pltpu

ALPHA = 0.1
Q = 1.0 - ALPHA
EPS = 1e-5
BLK = 400
LANES = 128


def _global_stats(part, n_elems):
    s = jnp.sum(part[:, 0, 0])
    ss = jnp.sum(part[:, 0, 1])
    mu = s / n_elems
    var = jnp.maximum(ss / n_elems - mu * mu, 0.0)
    return mu, jax.lax.rsqrt(var + EPS)


def _partial_vec(t):
    s = jnp.sum(t)
    ss = jnp.sum(t * t)
    lane = jax.lax.broadcasted_iota(jnp.int32, (1, 1, LANES), 2)
    return jnp.where(lane == 0, s, jnp.where(lane == 1, ss, 0.0))


def _pass1_body(x_ref, nb_ref, w1_ref, b1_ref,
                naggp_ref, h_ref, x1p_ref, part_ref):
    B, K, _, D = nb_ref.shape
    H1 = w1_ref.shape[1]
    xb = x_ref[...].reshape(B, D)
    h = jnp.dot(xb, w1_ref[...], preferred_element_type=jnp.float32) + b1_ref[...]
    nh = jnp.dot(nb_ref[...].reshape(B * K, D), w1_ref[...],
                 preferred_element_type=jnp.float32) + b1_ref[...]
    nh3 = nh.reshape(B, K, H1)
    x1p = h + Q * jnp.sum(nh3, axis=1)
    naggp_ref[...] = nh3 + Q * h[:, None, :]
    h_ref[...] = h
    x1p_ref[...] = x1p
    part_ref[...] = _partial_vec(x1p)


def _pass2_body(naggp_ref, h_ref, x1p_ref, part_ref, w2_ref, b2_ref, gb_ref,
                h2_ref, x3p_ref, t2_ref, part2_ref, *, n_elems):
    B, K, H1 = naggp_ref.shape
    H2 = w2_ref.shape[1]
    KH1 = float(K * H1)
    KH2 = float(K * H2)
    gamma = gb_ref[0, 0]
    beta = gb_ref[0, 1]
    mu_g, rs_g = _global_stats(part_ref[...], n_elems)
    h = h_ref[...]
    x1p = x1p_ref[...]
    x1 = jnp.maximum(gamma * (x1p - mu_g) * rs_g + beta, 0.0)
    naggp = naggp_ref[...]
    S = (x1p - h) * (1.0 / Q)
    sumvec = S + (Q * K) * h
    mu1 = (jnp.sum(sumvec, axis=-1) / KH1)[:, None, None]
    ss1 = (jnp.sum(naggp * naggp, axis=(1, 2)) / KH1)[:, None, None]
    var1 = jnp.maximum(ss1 - mu1 * mu1, 0.0)
    rs1 = jax.lax.rsqrt(var1 + EPS)
    nagg = jnp.maximum(gamma * (naggp - mu1) * rs1 + beta, 0.0)
    c = Q * x1 - (ALPHA * Q) * h
    n2 = Q * nagg + ALPHA * naggp + c[:, None, :]
    w2 = w2_ref[...]
    b2 = b2_ref[...]
    nh23 = (jnp.dot(n2.reshape(B * K, H1), w2,
                    preferred_element_type=jnp.float32)
            + b2).reshape(B, K, H2)
    s2 = jnp.sum(nh23, axis=1)
    m = (Q * (1.0 - K)) * x1 + ALPHA * h - ALPHA * S
    h2 = jnp.dot(m, w2, preferred_element_type=jnp.float32) + s2 + (1.0 - K) * b2
    x3p = h2 + Q * s2
    nagg2p = nh23 + Q * h2[:, None, :]
    sumvec2 = s2 + (Q * K) * h2
    mu2 = (jnp.sum(sumvec2, axis=-1) / KH2)[:, None, None]
    ss2 = (jnp.sum(nagg2p * nagg2p, axis=(1, 2)) / KH2)[:, None, None]
    var2 = jnp.maximum(ss2 - mu2 * mu2, 0.0)
    rs2 = jax.lax.rsqrt(var2 + EPS)
    nagg2 = jnp.maximum(gamma * (nagg2p - mu2) * rs2 + beta, 0.0)
    t2 = jnp.sum(nagg2, axis=1)
    h2_ref[...] = h2
    x3p_ref[...] = x3p
    t2_ref[...] = t2
    part2_ref[...] = _partial_vec(x3p)


def _pass3_body(h2_ref, x3p_ref, t2_ref, part_ref, wc_ref, bc_ref, gb_ref,
                out_ref, *, n_elems):
    gamma = gb_ref[0, 0]
    beta = gb_ref[0, 1]
    mu, rs = _global_stats(part_ref[...], n_elems)
    x3 = jnp.maximum(gamma * (x3p_ref[...] - mu) * rs + beta, 0.0)
    x4 = Q * (x3 + t2_ref[...]) + ALPHA * h2_ref[...]
    x4 = jnp.where(jnp.isnan(x4), 0.0, x4)
    out_ref[...] = jnp.dot(x4, wc_ref[...],
                           preferred_element_type=jnp.float32) + bc_ref[...]


def kernel(x, neighbor, W1, b1, W2, b2, Wc, bc, gamma, beta):
    N, _, D = x.shape
    K = neighbor.shape[1]
    H1 = W1.shape[1]
    H2 = W2.shape[1]
    C = Wc.shape[1]
    B = BLK
    G = N // B
    f32 = jnp.float32

    b1r = b1.reshape(1, H1)
    b2r = b2.reshape(1, H2)
    bcr = bc.reshape(1, C)
    gb = jnp.concatenate([gamma, beta]).reshape(1, 2)

    params = pltpu.CompilerParams(dimension_semantics=("parallel",))

    naggp, h, x1p, part1 = pl.pallas_call(
        _pass1_body,
        grid=(G,),
        in_specs=[
            pl.BlockSpec((B, 1, D), lambda i: (i, 0, 0)),
            pl.BlockSpec((B, K, 1, D), lambda i: (i, 0, 0, 0)),
            pl.BlockSpec((D, H1), lambda i: (0, 0)),
            pl.BlockSpec((1, H1), lambda i: (0, 0)),
        ],
        out_specs=[
            pl.BlockSpec((B, K, H1), lambda i: (i, 0, 0)),
            pl.BlockSpec((B, H1), lambda i: (i, 0)),
            pl.BlockSpec((B, H1), lambda i: (i, 0)),
            pl.BlockSpec((1, 1, LANES), lambda i: (i, 0, 0)),
        ],
        out_shape=[
            jax.ShapeDtypeStruct((N, K, H1), f32),
            jax.ShapeDtypeStruct((N, H1), f32),
            jax.ShapeDtypeStruct((N, H1), f32),
            jax.ShapeDtypeStruct((G, 1, LANES), f32),
        ],
        compiler_params=params,
    )(x, neighbor, W1, b1r)

    h2, x3p, t2, part2 = pl.pallas_call(
        functools.partial(_pass2_body, n_elems=float(N * H1)),
        grid=(G,),
        in_specs=[
            pl.BlockSpec((B, K, H1), lambda i: (i, 0, 0)),
            pl.BlockSpec((B, H1), lambda i: (i, 0)),
            pl.BlockSpec((B, H1), lambda i: (i, 0)),
            pl.BlockSpec((G, 1, LANES), lambda i: (0, 0, 0)),
            pl.BlockSpec((H1, H2), lambda i: (0, 0)),
            pl.BlockSpec((1, H2), lambda i: (0, 0)),
            pl.BlockSpec((1, 2), lambda i: (0, 0)),
        ],
        out_specs=[
            pl.BlockSpec((B, H2), lambda i: (i, 0)),
            pl.BlockSpec((B, H2), lambda i: (i, 0)),
            pl.BlockSpec((B, H2), lambda i: (i, 0)),
            pl.BlockSpec((1, 1, LANES), lambda i: (i, 0, 0)),
        ],
        out_shape=[
            jax.ShapeDtypeStruct((N, H2), f32),
            jax.ShapeDtypeStruct((N, H2), f32),
            jax.ShapeDtypeStruct((N, H2), f32),
            jax.ShapeDtypeStruct((G, 1, LANES), f32),
        ],
        compiler_params=params,
    )(naggp, h, x1p, part1, W2, b2r, gb)

    out = pl.pallas_call(
        functools.partial(_pass3_body, n_elems=float(N * H2)),
        grid=(1,),
        in_specs=[
            pl.BlockSpec((N, H2), lambda i: (0, 0)),
            pl.BlockSpec((N, H2), lambda i: (0, 0)),
            pl.BlockSpec((N, H2), lambda i: (0, 0)),
            pl.BlockSpec((G, 1, LANES), lambda i: (0, 0, 0)),
            pl.BlockSpec((H2, C), lambda i: (0, 0)),
            pl.BlockSpec((1, C), lambda i: (0, 0)),
            pl.BlockSpec((1, 2), lambda i: (0, 0)),
        ],
        out_specs=pl.BlockSpec((N, C), lambda i: (0, 0)),
        out_shape=jax.ShapeDtypeStruct((N, C), f32),
        compiler_params=params,
    )(h2, x3p, t2, part2, Wc, bcr, gb)

    return out


# speedup vs baseline: 2.8943x; 1.1010x over previous
import functools

import jax
import jax.numpy as jnp
from jax.experimental import pallas as pl
from jax.experimental.pallas import tpu as pltpu

ALPHA = 0.1
Q = 1.0 - ALPHA
EPS = 1e-5
BLK = 400
LANES = 128


def _global_stats(part, n_elems):
    s = jnp.sum(part[:, 0, 0])
    ss = jnp.sum(part[:, 0, 1])
    mu = s / n_elems
    var = jnp.maximum(ss / n_elems - mu * mu, 0.0)
    return mu, jax.lax.rsqrt(var + EPS)


def _partial_vec(t):
    s = jnp.sum(t)
    ss = jnp.sum(t * t)
    lane = jax.lax.broadcasted_iota(jnp.int32, (1, 1, LANES), 2)
    return jnp.where(lane == 0, s, jnp.where(lane == 1, ss, 0.0))


def _pass1_body(x_ref, nb_ref, w1_ref, b1_ref,
                naggp_ref, h_ref, x1p_ref, part_ref):
    B, K, D = nb_ref.shape
    H1 = w1_ref.shape[1]
    xb = x_ref[...]
    h = jnp.dot(xb, w1_ref[...], preferred_element_type=jnp.float32) + b1_ref[...]
    nh = jnp.dot(nb_ref[...].reshape(B * K, D), w1_ref[...],
                 preferred_element_type=jnp.float32) + b1_ref[...]
    nh3 = nh.reshape(B, K, H1)
    x1p = h + Q * jnp.sum(nh3, axis=1)
    naggp_ref[...] = (nh3 + Q * h[:, None, :]).astype(jnp.bfloat16)
    h_ref[...] = h
    x1p_ref[...] = x1p
    part_ref[...] = _partial_vec(x1p)


def _pass2_body(naggp_ref, h_ref, x1p_ref, part_ref, w2_ref, b2_ref, gb_ref,
                h2_ref, x3p_ref, t2_ref, part2_ref, *, n_elems):
    B, K, H1 = naggp_ref.shape
    H2 = w2_ref.shape[1]
    KH1 = float(K * H1)
    KH2 = float(K * H2)
    gamma = gb_ref[0, 0]
    beta = gb_ref[0, 1]
    mu_g, rs_g = _global_stats(part_ref[...], n_elems)
    h = h_ref[...]
    x1p = x1p_ref[...]
    x1 = jnp.maximum(gamma * (x1p - mu_g) * rs_g + beta, 0.0)
    naggp = naggp_ref[...].astype(jnp.float32)
    S = (x1p - h) * (1.0 / Q)
    sumvec = S + (Q * K) * h
    mu1 = (jnp.sum(sumvec, axis=-1) / KH1)[:, None, None]
    d1 = naggp - mu1
    var1 = (jnp.sum(d1 * d1, axis=(1, 2)) / KH1)[:, None, None]
    rs1 = jax.lax.rsqrt(var1 + EPS)
    nagg = jnp.maximum(gamma * d1 * rs1 + beta, 0.0)
    c = Q * x1 - (ALPHA * Q) * h
    n2 = Q * nagg + ALPHA * naggp + c[:, None, :]
    w2 = w2_ref[...]
    b2 = b2_ref[...]
    nh23 = (jnp.dot(n2.reshape(B * K, H1), w2,
                    preferred_element_type=jnp.float32)
            + b2).reshape(B, K, H2)
    s2 = jnp.sum(nh23, axis=1)
    m = (Q * (1.0 - K)) * x1 + ALPHA * h - ALPHA * S
    h2 = jnp.dot(m, w2, preferred_element_type=jnp.float32) + s2 + (1.0 - K) * b2
    x3p = h2 + Q * s2
    nagg2p = nh23 + Q * h2[:, None, :]
    sumvec2 = s2 + (Q * K) * h2
    mu2 = (jnp.sum(sumvec2, axis=-1) / KH2)[:, None, None]
    d2 = nagg2p - mu2
    var2 = (jnp.sum(d2 * d2, axis=(1, 2)) / KH2)[:, None, None]
    rs2 = jax.lax.rsqrt(var2 + EPS)
    nagg2 = jnp.maximum(gamma * d2 * rs2 + beta, 0.0)
    t2 = jnp.sum(nagg2, axis=1)
    h2_ref[...] = h2
    x3p_ref[...] = x3p
    t2_ref[...] = t2
    part2_ref[...] = _partial_vec(x3p)


def _pass3_body(h2_ref, x3p_ref, t2_ref, part_ref, wc_ref, bc_ref, gb_ref,
                out_ref, *, n_elems):
    gamma = gb_ref[0, 0]
    beta = gb_ref[0, 1]
    mu, rs = _global_stats(part_ref[...], n_elems)
    x3 = jnp.maximum(gamma * (x3p_ref[...] - mu) * rs + beta, 0.0)
    x4 = Q * (x3 + t2_ref[...]) + ALPHA * h2_ref[...]
    x4 = jnp.where(jnp.isnan(x4), 0.0, x4)
    out_ref[...] = jnp.dot(x4, wc_ref[...],
                           preferred_element_type=jnp.float32) + bc_ref[...]


def kernel(x, neighbor, W1, b1, W2, b2, Wc, bc, gamma, beta):
    N, _, D = x.shape
    K = neighbor.shape[1]
    H1 = W1.shape[1]
    H2 = W2.shape[1]
    C = Wc.shape[1]
    B = BLK
    G = N // B
    f32 = jnp.float32

    x2d = x.reshape(N, D)
    nb3 = neighbor.reshape(N, K, D)
    b1r = b1.reshape(1, H1)
    b2r = b2.reshape(1, H2)
    bcr = bc.reshape(1, C)
    gb = jnp.concatenate([gamma, beta]).reshape(1, 2)

    params = pltpu.CompilerParams(dimension_semantics=("parallel",))

    naggp, h, x1p, part1 = pl.pallas_call(
        _pass1_body,
        grid=(G,),
        in_specs=[
            pl.BlockSpec((B, D), lambda i: (i, 0)),
            pl.BlockSpec((B, K, D), lambda i: (i, 0, 0)),
            pl.BlockSpec((D, H1), lambda i: (0, 0)),
            pl.BlockSpec((1, H1), lambda i: (0, 0)),
        ],
        out_specs=[
            pl.BlockSpec((B, K, H1), lambda i: (i, 0, 0)),
            pl.BlockSpec((B, H1), lambda i: (i, 0)),
            pl.BlockSpec((B, H1), lambda i: (i, 0)),
            pl.BlockSpec((1, 1, LANES), lambda i: (i, 0, 0)),
        ],
        out_shape=[
            jax.ShapeDtypeStruct((N, K, H1), jnp.bfloat16),
            jax.ShapeDtypeStruct((N, H1), f32),
            jax.ShapeDtypeStruct((N, H1), f32),
            jax.ShapeDtypeStruct((G, 1, LANES), f32),
        ],
        compiler_params=params,
    )(x2d, nb3, W1, b1r)

    h2, x3p, t2, part2 = pl.pallas_call(
        functools.partial(_pass2_body, n_elems=float(N * H1)),
        grid=(G,),
        in_specs=[
            pl.BlockSpec((B, K, H1), lambda i: (i, 0, 0)),
            pl.BlockSpec((B, H1), lambda i: (i, 0)),
            pl.BlockSpec((B, H1), lambda i: (i, 0)),
            pl.BlockSpec((G, 1, LANES), lambda i: (0, 0, 0)),
            pl.BlockSpec((H1, H2), lambda i: (0, 0)),
            pl.BlockSpec((1, H2), lambda i: (0, 0)),
            pl.BlockSpec((1, 2), lambda i: (0, 0)),
        ],
        out_specs=[
            pl.BlockSpec((B, H2), lambda i: (i, 0)),
            pl.BlockSpec((B, H2), lambda i: (i, 0)),
            pl.BlockSpec((B, H2), lambda i: (i, 0)),
            pl.BlockSpec((1, 1, LANES), lambda i: (i, 0, 0)),
        ],
        out_shape=[
            jax.ShapeDtypeStruct((N, H2), f32),
            jax.ShapeDtypeStruct((N, H2), f32),
            jax.ShapeDtypeStruct((N, H2), f32),
            jax.ShapeDtypeStruct((G, 1, LANES), f32),
        ],
        compiler_params=params,
    )(naggp, h, x1p, part1, W2, b2r, gb)

    out = pl.pallas_call(
        functools.partial(_pass3_body, n_elems=float(N * H2)),
        grid=(1,),
        in_specs=[
            pl.BlockSpec((N, H2), lambda i: (0, 0)),
            pl.BlockSpec((N, H2), lambda i: (0, 0)),
            pl.BlockSpec((N, H2), lambda i: (0, 0)),
            pl.BlockSpec((G, 1, LANES), lambda i: (0, 0, 0)),
            pl.BlockSpec((H2, C), lambda i: (0, 0)),
            pl.BlockSpec((1, C), lambda i: (0, 0)),
            pl.BlockSpec((1, 2), lambda i: (0, 0)),
        ],
        out_specs=pl.BlockSpec((N, C), lambda i: (0, 0)),
        out_shape=jax.ShapeDtypeStruct((N, C), f32),
        compiler_params=params,
    )(h2, x3p, t2, part2, Wc, bcr, gb)

    return out


# speedup vs baseline: 2.9968x; 1.0354x over previous
import functools

import jax
import jax.numpy as jnp
from jax.experimental import pallas as pl
from jax.experimental.pallas import tpu as pltpu

ALPHA = 0.1
Q = 1.0 - ALPHA
EPS = 1e-5
BLK = 400
LANES = 128


def _global_stats(part, n_elems):
    s = jnp.sum(part[:, 0, 0])
    ss = jnp.sum(part[:, 0, 1])
    mu = s / n_elems
    var = jnp.maximum(ss / n_elems - mu * mu, 0.0)
    return mu, jax.lax.rsqrt(var + EPS)


def _partial_vec(t):
    s = jnp.sum(t)
    ss = jnp.sum(t * t)
    lane = jax.lax.broadcasted_iota(jnp.int32, (1, 1, LANES), 2)
    return jnp.where(lane == 0, s, jnp.where(lane == 1, ss, 0.0))


def _ksum(v):
    while v.shape[1] > 8:
        half = v.shape[1] // 2
        v = v[:, :half, :] + v[:, half:, :]
    return jnp.sum(v, axis=1)


def _pass1_body(x_ref, nb_ref, w1_ref, b1_ref,
                nh_ref, h_ref, x1p_ref, part_ref):
    B, K, D = nb_ref.shape
    H1 = w1_ref.shape[1]
    xb = x_ref[...]
    h = jnp.dot(xb, w1_ref[...], preferred_element_type=jnp.float32) + b1_ref[...]
    nh = jnp.dot(nb_ref[...].reshape(B * K, D), w1_ref[...],
                 preferred_element_type=jnp.float32) + b1_ref[...]
    nh3 = nh.reshape(B, K, H1)
    x1p = h + Q * _ksum(nh3)
    nh_ref[...] = nh3.astype(jnp.bfloat16)
    h_ref[...] = h
    x1p_ref[...] = x1p
    part_ref[...] = _partial_vec(x1p)


def _pass2_body(nh_ref, h_ref, x1p_ref, part_ref, w2_ref, b2_ref, gb_ref,
                h2_ref, x3p_ref, t2_ref, part2_ref, *, n_elems):
    B, K, H1 = nh_ref.shape
    H2 = w2_ref.shape[1]
    KH1 = float(K * H1)
    KH2 = float(K * H2)
    gamma = gb_ref[0, 0]
    beta = gb_ref[0, 1]
    mu_g, rs_g = _global_stats(part_ref[...], n_elems)
    h = h_ref[...]
    x1p = x1p_ref[...]
    x1 = jnp.maximum(gamma * (x1p - mu_g) * rs_g + beta, 0.0)
    nh = nh_ref[...].astype(jnp.float32)
    S = (x1p - h) * (1.0 / Q)
    sumvec = S + (Q * K) * h
    mu1 = (jnp.sum(sumvec, axis=-1) / KH1)[:, None, None]
    d1 = nh - (mu1 - Q * h[:, None, :])
    rows1 = jnp.sum(d1 * d1, axis=2)
    var1 = (jnp.sum(rows1, axis=1) / KH1)[:, None, None]
    g1 = gamma * jax.lax.rsqrt(var1 + EPS)
    nagg = jnp.maximum(d1 * g1 + beta, 0.0)
    n2 = Q * nagg + ALPHA * nh + (Q * x1)[:, None, :]
    w2 = w2_ref[...]
    b2 = b2_ref[...]
    nh23 = (jnp.dot(n2.reshape(B * K, H1), w2,
                    preferred_element_type=jnp.float32)
            + b2).reshape(B, K, H2)
    s2 = _ksum(nh23)
    m = (Q * (1.0 - K)) * x1 + ALPHA * h - ALPHA * S
    h2 = jnp.dot(m, w2, preferred_element_type=jnp.float32) + s2 + (1.0 - K) * b2
    x3p = h2 + Q * s2
    sumvec2 = s2 + (Q * K) * h2
    mu2 = (jnp.sum(sumvec2, axis=-1) / KH2)[:, None, None]
    d2 = nh23 - (mu2 - Q * h2[:, None, :])
    rows2 = jnp.sum(d2 * d2, axis=2)
    var2 = (jnp.sum(rows2, axis=1) / KH2)[:, None, None]
    g2 = gamma * jax.lax.rsqrt(var2 + EPS)
    nagg2 = jnp.maximum(d2 * g2 + beta, 0.0)
    t2 = _ksum(nagg2)
    h2_ref[...] = h2
    x3p_ref[...] = x3p
    t2_ref[...] = t2
    part2_ref[...] = _partial_vec(x3p)


def _pass3_body(h2_ref, x3p_ref, t2_ref, part_ref, wc_ref, bc_ref, gb_ref,
                out_ref, *, n_elems):
    gamma = gb_ref[0, 0]
    beta = gb_ref[0, 1]
    mu, rs = _global_stats(part_ref[...], n_elems)
    x3 = jnp.maximum(gamma * (x3p_ref[...] - mu) * rs + beta, 0.0)
    x4 = Q * (x3 + t2_ref[...]) + ALPHA * h2_ref[...]
    x4 = jnp.where(jnp.isnan(x4), 0.0, x4)
    out_ref[...] = jnp.dot(x4, wc_ref[...],
                           preferred_element_type=jnp.float32) + bc_ref[...]


def kernel(x, neighbor, W1, b1, W2, b2, Wc, bc, gamma, beta):
    N, _, D = x.shape
    K = neighbor.shape[1]
    H1 = W1.shape[1]
    H2 = W2.shape[1]
    C = Wc.shape[1]
    B = BLK
    G = N // B
    f32 = jnp.float32

    x2d = x.reshape(N, D)
    nb3 = neighbor.reshape(N, K, D)
    b1r = b1.reshape(1, H1)
    b2r = b2.reshape(1, H2)
    bcr = bc.reshape(1, C)
    gb = jnp.concatenate([gamma, beta]).reshape(1, 2)

    params = pltpu.CompilerParams(dimension_semantics=("parallel",))

    nh, h, x1p, part1 = pl.pallas_call(
        _pass1_body,
        grid=(G,),
        in_specs=[
            pl.BlockSpec((B, D), lambda i: (i, 0)),
            pl.BlockSpec((B, K, D), lambda i: (i, 0, 0)),
            pl.BlockSpec((D, H1), lambda i: (0, 0)),
            pl.BlockSpec((1, H1), lambda i: (0, 0)),
        ],
        out_specs=[
            pl.BlockSpec((B, K, H1), lambda i: (i, 0, 0)),
            pl.BlockSpec((B, H1), lambda i: (i, 0)),
            pl.BlockSpec((B, H1), lambda i: (i, 0)),
            pl.BlockSpec((1, 1, LANES), lambda i: (i, 0, 0)),
        ],
        out_shape=[
            jax.ShapeDtypeStruct((N, K, H1), jnp.bfloat16),
            jax.ShapeDtypeStruct((N, H1), f32),
            jax.ShapeDtypeStruct((N, H1), f32),
            jax.ShapeDtypeStruct((G, 1, LANES), f32),
        ],
        compiler_params=params,
    )(x2d, nb3, W1, b1r)

    h2, x3p, t2, part2 = pl.pallas_call(
        functools.partial(_pass2_body, n_elems=float(N * H1)),
        grid=(G,),
        in_specs=[
            pl.BlockSpec((B, K, H1), lambda i: (i, 0, 0)),
            pl.BlockSpec((B, H1), lambda i: (i, 0)),
            pl.BlockSpec((B, H1), lambda i: (i, 0)),
            pl.BlockSpec((G, 1, LANES), lambda i: (0, 0, 0)),
            pl.BlockSpec((H1, H2), lambda i: (0, 0)),
            pl.BlockSpec((1, H2), lambda i: (0, 0)),
            pl.BlockSpec((1, 2), lambda i: (0, 0)),
        ],
        out_specs=[
            pl.BlockSpec((B, H2), lambda i: (i, 0)),
            pl.BlockSpec((B, H2), lambda i: (i, 0)),
            pl.BlockSpec((B, H2), lambda i: (i, 0)),
            pl.BlockSpec((1, 1, LANES), lambda i: (i, 0, 0)),
        ],
        out_shape=[
            jax.ShapeDtypeStruct((N, H2), f32),
            jax.ShapeDtypeStruct((N, H2), f32),
            jax.ShapeDtypeStruct((N, H2), f32),
            jax.ShapeDtypeStruct((G, 1, LANES), f32),
        ],
        compiler_params=params,
    )(nh, h, x1p, part1, W2, b2r, gb)

    out = pl.pallas_call(
        functools.partial(_pass3_body, n_elems=float(N * H2)),
        grid=(1,),
        in_specs=[
            pl.BlockSpec((N, H2), lambda i: (0, 0)),
            pl.BlockSpec((N, H2), lambda i: (0, 0)),
            pl.BlockSpec((N, H2), lambda i: (0, 0)),
            pl.BlockSpec((G, 1, LANES), lambda i: (0, 0, 0)),
            pl.BlockSpec((H2, C), lambda i: (0, 0)),
            pl.BlockSpec((1, C), lambda i: (0, 0)),
            pl.BlockSpec((1, 2), lambda i: (0, 0)),
        ],
        out_specs=pl.BlockSpec((N, C), lambda i: (0, 0)),
        out_shape=jax.ShapeDtypeStruct((N, C), f32),
        compiler_params=params,
    )(h2, x3p, t2, part2, Wc, bcr, gb)

    return out


# speedup vs baseline: 3.8760x; 1.2934x over previous
import functools

import jax
import jax.numpy as jnp
from jax.experimental import pallas as pl
from jax.experimental.pallas import tpu as pltpu
from jax.scipy.linalg import block_diag

ALPHA = 0.1
Q = 1.0 - ALPHA
EPS = 1e-5
BLK = 400
LANES = 128
P = 4


def _global_stats(part, n_elems):
    s = jnp.sum(part[:, 0, 0])
    ss = jnp.sum(part[:, 0, 1])
    mu = s / n_elems
    var = jnp.maximum(ss / n_elems - mu * mu, 0.0)
    return mu, jax.lax.rsqrt(var + EPS)


def _partial_vec(t):
    s = jnp.sum(t)
    ss = jnp.sum(t * t)
    lane = jax.lax.broadcasted_iota(jnp.int32, (1, 1, LANES), 2)
    return jnp.where(lane == 0, s, jnp.where(lane == 1, ss, 0.0))


def _fold(v, width):
    while v.shape[-1] > width:
        half = v.shape[-1] // 2
        v = v[:, :half] + v[:, half:]
    return v


def _pack4(v):
    K = v.shape[1]
    v = jnp.concatenate([v[:, : K // 2], v[:, K // 2:]], axis=-1)
    v = jnp.concatenate([v[:, : K // 4], v[:, K // 4:]], axis=-1)
    return v


def _pass1_body(x_ref, nb_ref, w1_ref, b1_ref,
                nh_ref, h_ref, x1p_ref, part_ref):
    B, K, D = nb_ref.shape
    H1 = w1_ref.shape[1]
    xb = x_ref[...]
    h = jnp.dot(xb, w1_ref[...], preferred_element_type=jnp.float32) + b1_ref[...]
    nh = jnp.dot(nb_ref[...].reshape(B * K, D), w1_ref[...],
                 preferred_element_type=jnp.float32) + b1_ref[...]
    nhp = _pack4(nh.reshape(B, K, H1))
    x1p = h + Q * _fold(jnp.sum(nhp, axis=1), H1)
    nh_ref[...] = nhp.astype(jnp.bfloat16)
    h_ref[...] = h
    x1p_ref[...] = x1p
    part_ref[...] = _partial_vec(x1p)


def _pass2_body(nh_ref, h_ref, x1p_ref, part_ref, w2d_ref, b2d_ref, gb_ref,
                h2_ref, x3p_ref, t2_ref, part2_ref, *, n_elems, K):
    B, R, PH1 = nh_ref.shape
    H1 = PH1 // P
    H2 = w2d_ref.shape[1] // P
    KH1 = float(K * H1)
    KH2 = float(K * H2)
    gamma = gb_ref[0, 0]
    beta = gb_ref[0, 1]
    mu_g, rs_g = _global_stats(part_ref[...], n_elems)
    h = h_ref[...]
    x1p = x1p_ref[...]
    x1 = jnp.maximum(gamma * (x1p - mu_g) * rs_g + beta, 0.0)
    nh = nh_ref[...].astype(jnp.float32)
    S = (x1p - h) * (1.0 / Q)
    sumvec = S + (Q * K) * h
    mu1 = (jnp.sum(sumvec, axis=-1) / KH1)[:, None, None]
    h4 = jnp.concatenate([h] * P, axis=-1)
    d1 = nh - (mu1 - Q * h4[:, None, :])
    rows1 = jnp.sum(d1 * d1, axis=2)
    var1 = (jnp.sum(rows1, axis=1) / KH1)[:, None, None]
    g1 = gamma * jax.lax.rsqrt(var1 + EPS)
    nagg = jnp.maximum(d1 * g1 + beta, 0.0)
    x14 = jnp.concatenate([x1] * P, axis=-1)
    n2 = Q * nagg + ALPHA * nh + (Q * x14)[:, None, :]
    nh23 = (jnp.dot(n2.reshape(B * R, PH1), w2d_ref[...],
                    preferred_element_type=jnp.float32)
            + b2d_ref[...]).reshape(B, R, P * H2)
    s2 = _fold(jnp.sum(nh23, axis=1), H2)
    m = (Q * (1.0 - K)) * x1 + ALPHA * h - ALPHA * S
    w2 = w2d_ref[:H1, :H2]
    b2 = b2d_ref[:, :H2]
    h2 = jnp.dot(m, w2, preferred_element_type=jnp.float32) + s2 + (1.0 - K) * b2
    x3p = h2 + Q * s2
    sumvec2 = s2 + (Q * K) * h2
    mu2 = (jnp.sum(sumvec2, axis=-1) / KH2)[:, None, None]
    h24 = jnp.concatenate([h2] * P, axis=-1)
    d2 = nh23 - (mu2 - Q * h24[:, None, :])
    rows2 = jnp.sum(d2 * d2, axis=2)
    var2 = (jnp.sum(rows2, axis=1) / KH2)[:, None, None]
    g2 = gamma * jax.lax.rsqrt(var2 + EPS)
    nagg2 = jnp.maximum(d2 * g2 + beta, 0.0)
    t2 = _fold(jnp.sum(nagg2, axis=1), H2)
    h2_ref[...] = h2
    x3p_ref[...] = x3p
    t2_ref[...] = t2
    part2_ref[...] = _partial_vec(x3p)


def _pass3_body(h2_ref, x3p_ref, t2_ref, part_ref, wc_ref, bc_ref, gb_ref,
                out_ref, *, n_elems):
    gamma = gb_ref[0, 0]
    beta = gb_ref[0, 1]
    mu, rs = _global_stats(part_ref[...], n_elems)
    x3 = jnp.maximum(gamma * (x3p_ref[...] - mu) * rs + beta, 0.0)
    x4 = Q * (x3 + t2_ref[...]) + ALPHA * h2_ref[...]
    x4 = jnp.where(jnp.isnan(x4), 0.0, x4)
    out_ref[...] = jnp.dot(x4, wc_ref[...],
                           preferred_element_type=jnp.float32) + bc_ref[...]


def kernel(x, neighbor, W1, b1, W2, b2, Wc, bc, gamma, beta):
    N, _, D = x.shape
    K = neighbor.shape[1]
    H1 = W1.shape[1]
    H2 = W2.shape[1]
    C = Wc.shape[1]
    B = BLK
    G = N // B
    R = K // P
    f32 = jnp.float32

    x2d = x.reshape(N, D)
    nb3 = neighbor.reshape(N, K, D)
    b1r = b1.reshape(1, H1)
    bcr = bc.reshape(1, C)
    gb = jnp.concatenate([gamma, beta]).reshape(1, 2)
    W2d = block_diag(*([W2] * P))
    b2d = jnp.concatenate([b2] * P).reshape(1, P * H2)

    params = pltpu.CompilerParams(dimension_semantics=("parallel",))

    nh, h, x1p, part1 = pl.pallas_call(
        _pass1_body,
        grid=(G,),
        in_specs=[
            pl.BlockSpec((B, D), lambda i: (i, 0)),
            pl.BlockSpec((B, K, D), lambda i: (i, 0, 0)),
            pl.BlockSpec((D, H1), lambda i: (0, 0)),
            pl.BlockSpec((1, H1), lambda i: (0, 0)),
        ],
        out_specs=[
            pl.BlockSpec((B, R, P * H1), lambda i: (i, 0, 0)),
            pl.BlockSpec((B, H1), lambda i: (i, 0)),
            pl.BlockSpec((B, H1), lambda i: (i, 0)),
            pl.BlockSpec((1, 1, LANES), lambda i: (i, 0, 0)),
        ],
        out_shape=[
            jax.ShapeDtypeStruct((N, R, P * H1), jnp.bfloat16),
            jax.ShapeDtypeStruct((N, H1), f32),
            jax.ShapeDtypeStruct((N, H1), f32),
            jax.ShapeDtypeStruct((G, 1, LANES), f32),
        ],
        compiler_params=params,
    )(x2d, nb3, W1, b1r)

    h2, x3p, t2, part2 = pl.pallas_call(
        functools.partial(_pass2_body, n_elems=float(N * H1), K=K),
        grid=(G,),
        in_specs=[
            pl.BlockSpec((B, R, P * H1), lambda i: (i, 0, 0)),
            pl.BlockSpec((B, H1), lambda i: (i, 0)),
            pl.BlockSpec((B, H1), lambda i: (i, 0)),
            pl.BlockSpec((G, 1, LANES), lambda i: (0, 0, 0)),
            pl.BlockSpec((P * H1, P * H2), lambda i: (0, 0)),
            pl.BlockSpec((1, P * H2), lambda i: (0, 0)),
            pl.BlockSpec((1, 2), lambda i: (0, 0)),
        ],
        out_specs=[
            pl.BlockSpec((B, H2), lambda i: (i, 0)),
            pl.BlockSpec((B, H2), lambda i: (i, 0)),
            pl.BlockSpec((B, H2), lambda i: (i, 0)),
            pl.BlockSpec((1, 1, LANES), lambda i: (i, 0, 0)),
        ],
        out_shape=[
            jax.ShapeDtypeStruct((N, H2), f32),
            jax.ShapeDtypeStruct((N, H2), f32),
            jax.ShapeDtypeStruct((N, H2), f32),
            jax.ShapeDtypeStruct((G, 1, LANES), f32),
        ],
        compiler_params=params,
    )(nh, h, x1p, part1, W2d, b2d, gb)

    out = pl.pallas_call(
        functools.partial(_pass3_body, n_elems=float(N * H2)),
        grid=(1,),
        in_specs=[
            pl.BlockSpec((N, H2), lambda i: (0, 0)),
            pl.BlockSpec((N, H2), lambda i: (0, 0)),
            pl.BlockSpec((N, H2), lambda i: (0, 0)),
            pl.BlockSpec((G, 1, LANES), lambda i: (0, 0, 0)),
            pl.BlockSpec((H2, C), lambda i: (0, 0)),
            pl.BlockSpec((1, C), lambda i: (0, 0)),
            pl.BlockSpec((1, 2), lambda i: (0, 0)),
        ],
        out_specs=pl.BlockSpec((N, C), lambda i: (0, 0)),
        out_shape=jax.ShapeDtypeStruct((N, C), f32),
        compiler_params=params,
    )(h2, x3p, t2, part2, Wc, bcr, gb)

    return out


# speedup vs baseline: 3.9515x; 1.0195x over previous
import functools

import jax
import jax.numpy as jnp
from jax.experimental import pallas as pl
from jax.experimental.pallas import tpu as pltpu
from jax.scipy.linalg import block_diag

ALPHA = 0.1
Q = 1.0 - ALPHA
EPS = 1e-5
BLK = 400
LANES = 128
P = 4


def _global_stats(part, n_elems):
    s = jnp.sum(part[:, 0, 0])
    ss = jnp.sum(part[:, 0, 1])
    mu = s / n_elems
    var = jnp.maximum(ss / n_elems - mu * mu, 0.0)
    return mu, jax.lax.rsqrt(var + EPS)


def _partial_vec(t):
    s = jnp.sum(t)
    ss = jnp.sum(t * t)
    lane = jax.lax.broadcasted_iota(jnp.int32, (1, 1, LANES), 2)
    return jnp.where(lane == 0, s, jnp.where(lane == 1, ss, 0.0))


def _fold(v, width):
    while v.shape[-1] > width:
        half = v.shape[-1] // 2
        v = v[:, :half] + v[:, half:]
    return v


def _pack4(v):
    K = v.shape[1]
    v = jnp.concatenate([v[:, : K // 2], v[:, K // 2:]], axis=-1)
    v = jnp.concatenate([v[:, : K // 4], v[:, K // 4:]], axis=-1)
    return v


def _pass1_body(x_ref, nb_ref, w1_ref, b1_ref,
                nh_ref, h_ref, x1p_ref, part_ref):
    B, K, D = nb_ref.shape
    H1 = w1_ref.shape[1]
    xb = x_ref[...]
    h = jnp.dot(xb, w1_ref[...], preferred_element_type=jnp.float32) + b1_ref[...]
    nh = jnp.dot(nb_ref[...].reshape(B * K, D), w1_ref[...],
                 preferred_element_type=jnp.float32) + b1_ref[...]
    nhp = _pack4(nh.reshape(B, K, H1))
    x1p = h + Q * _fold(jnp.sum(nhp, axis=1), H1)
    nh_ref[...] = nhp.astype(jnp.bfloat16)
    h_ref[...] = h
    x1p_ref[...] = x1p
    part_ref[...] = _partial_vec(x1p)


def _pass2_body(nh_ref, h_ref, x1p_ref, part_ref, w2d_ref, b2d_ref, gb_ref,
                h2_ref, x3p_ref, t2_ref, part2_ref, *, n_elems, K):
    B, R, PH1 = nh_ref.shape
    H1 = PH1 // P
    H2 = w2d_ref.shape[1] // P
    KH1 = float(K * H1)
    KH2 = float(K * H2)
    gamma = gb_ref[0, 0]
    beta = gb_ref[0, 1]
    mu_g, rs_g = _global_stats(part_ref[...], n_elems)
    h = h_ref[...]
    x1p = x1p_ref[...]
    x1 = jnp.maximum(gamma * (x1p - mu_g) * rs_g + beta, 0.0)
    nh = nh_ref[...].astype(jnp.float32)
    S = (x1p - h) * (1.0 / Q)
    sumvec = S + (Q * K) * h
    mu1 = (jnp.sum(sumvec, axis=-1) / KH1)[:, None, None]
    h4 = jnp.concatenate([h] * P, axis=-1)
    d1 = nh - (mu1 - Q * h4[:, None, :])
    rows1 = jnp.sum(d1 * d1, axis=2)
    var1 = (jnp.sum(rows1, axis=1) / KH1)[:, None, None]
    g1 = gamma * jax.lax.rsqrt(var1 + EPS)
    nagg = jnp.maximum(d1 * g1 + beta, 0.0)
    x14 = jnp.concatenate([x1] * P, axis=-1)
    n2 = Q * nagg + ALPHA * nh + (Q * x14)[:, None, :]
    nh23 = (jnp.dot(n2.reshape(B * R, PH1), w2d_ref[...],
                    preferred_element_type=jnp.float32)
            + b2d_ref[...]).reshape(B, R, P * H2)
    s2 = _fold(jnp.sum(nh23, axis=1), H2)
    m = (Q * (1.0 - K)) * x1 + ALPHA * h - ALPHA * S
    w2 = w2d_ref[:H1, :H2]
    b2 = b2d_ref[:, :H2]
    h2 = jnp.dot(m, w2, preferred_element_type=jnp.float32) + s2 + (1.0 - K) * b2
    x3p = h2 + Q * s2
    sumvec2 = s2 + (Q * K) * h2
    mu2 = (jnp.sum(sumvec2, axis=-1) / KH2)[:, None, None]
    h24 = jnp.concatenate([h2] * P, axis=-1)
    d2 = nh23 - (mu2 - Q * h24[:, None, :])
    rows2 = jnp.sum(d2 * d2, axis=2)
    var2 = (jnp.sum(rows2, axis=1) / KH2)[:, None, None]
    g2 = gamma * jax.lax.rsqrt(var2 + EPS)
    nagg2 = jnp.maximum(d2 * g2 + beta, 0.0)
    t2 = _fold(jnp.sum(nagg2, axis=1), H2)
    h2_ref[...] = h2
    x3p_ref[...] = x3p
    t2_ref[...] = t2
    part2_ref[...] = _partial_vec(x3p)


def _pass3_body(h2_ref, x3p_ref, t2_ref, part_ref, wc_ref, bc_ref, gb_ref,
                out_ref, *, n_elems):
    gamma = gb_ref[0, 0]
    beta = gb_ref[0, 1]
    mu, rs = _global_stats(part_ref[...], n_elems)
    x3 = jnp.maximum(gamma * (x3p_ref[...] - mu) * rs + beta, 0.0)
    x4 = Q * (x3 + t2_ref[...]) + ALPHA * h2_ref[...]
    x4 = jnp.where(jnp.isnan(x4), 0.0, x4)
    out_ref[...] = jnp.dot(x4, wc_ref[...],
                           preferred_element_type=jnp.float32) + bc_ref[...]


def kernel(x, neighbor, W1, b1, W2, b2, Wc, bc, gamma, beta):
    N, _, D = x.shape
    K = neighbor.shape[1]
    H1 = W1.shape[1]
    H2 = W2.shape[1]
    C = Wc.shape[1]
    B = BLK
    G = N // B
    R = K // P
    f32 = jnp.float32

    x2d = x.reshape(N, D)
    nb3 = neighbor.reshape(N, K, D)
    b1r = b1.reshape(1, H1)
    bcr = bc.reshape(1, C)
    gb = jnp.concatenate([gamma, beta]).reshape(1, 2)
    W2d = block_diag(*([W2] * P))
    b2d = jnp.concatenate([b2] * P).reshape(1, P * H2)

    params = pltpu.CompilerParams(dimension_semantics=("parallel",))

    nh, h, x1p, part1 = pl.pallas_call(
        _pass1_body,
        grid=(G,),
        in_specs=[
            pl.BlockSpec((B, D), lambda i: (i, 0)),
            pl.BlockSpec((B, K, D), lambda i: (i, 0, 0)),
            pl.BlockSpec((D, H1), lambda i: (0, 0)),
            pl.BlockSpec((1, H1), lambda i: (0, 0)),
        ],
        out_specs=[
            pl.BlockSpec((B, R, P * H1), lambda i: (i, 0, 0)),
            pl.BlockSpec((B, H1), lambda i: (i, 0)),
            pl.BlockSpec((B, H1), lambda i: (i, 0)),
            pl.BlockSpec((1, 1, LANES), lambda i: (i, 0, 0)),
        ],
        out_shape=[
            jax.ShapeDtypeStruct((N, R, P * H1), jnp.bfloat16),
            jax.ShapeDtypeStruct((N, H1), f32),
            jax.ShapeDtypeStruct((N, H1), f32),
            jax.ShapeDtypeStruct((G, 1, LANES), f32),
        ],
        compiler_params=params,
    )(x2d, nb3, W1, b1r)

    B2 = 1000
    G2 = N // B2
    h2, x3p, t2, part2 = pl.pallas_call(
        functools.partial(_pass2_body, n_elems=float(N * H1), K=K),
        grid=(G2,),
        in_specs=[
            pl.BlockSpec((B2, R, P * H1), lambda i: (i, 0, 0)),
            pl.BlockSpec((B2, H1), lambda i: (i, 0)),
            pl.BlockSpec((B2, H1), lambda i: (i, 0)),
            pl.BlockSpec((G, 1, LANES), lambda i: (0, 0, 0)),
            pl.BlockSpec((P * H1, P * H2), lambda i: (0, 0)),
            pl.BlockSpec((1, P * H2), lambda i: (0, 0)),
            pl.BlockSpec((1, 2), lambda i: (0, 0)),
        ],
        out_specs=[
            pl.BlockSpec((B2, H2), lambda i: (i, 0)),
            pl.BlockSpec((B2, H2), lambda i: (i, 0)),
            pl.BlockSpec((B2, H2), lambda i: (i, 0)),
            pl.BlockSpec((1, 1, LANES), lambda i: (i, 0, 0)),
        ],
        out_shape=[
            jax.ShapeDtypeStruct((N, H2), f32),
            jax.ShapeDtypeStruct((N, H2), f32),
            jax.ShapeDtypeStruct((N, H2), f32),
            jax.ShapeDtypeStruct((G2, 1, LANES), f32),
        ],
        compiler_params=params,
    )(nh, h, x1p, part1, W2d, b2d, gb)

    out = pl.pallas_call(
        functools.partial(_pass3_body, n_elems=float(N * H2)),
        grid=(1,),
        in_specs=[
            pl.BlockSpec((N, H2), lambda i: (0, 0)),
            pl.BlockSpec((N, H2), lambda i: (0, 0)),
            pl.BlockSpec((N, H2), lambda i: (0, 0)),
            pl.BlockSpec((G2, 1, LANES), lambda i: (0, 0, 0)),
            pl.BlockSpec((H2, C), lambda i: (0, 0)),
            pl.BlockSpec((1, C), lambda i: (0, 0)),
            pl.BlockSpec((1, 2), lambda i: (0, 0)),
        ],
        out_specs=pl.BlockSpec((N, C), lambda i: (0, 0)),
        out_shape=jax.ShapeDtypeStruct((N, C), f32),
        compiler_params=params,
    )(h2, x3p, t2, part2, Wc, bcr, gb)

    return out
